# baseline (speedup 1.0000x reference)
import jax
import jax.numpy as jnp
from jax import lax
from jax.experimental import pallas as pl
from jax.experimental.pallas import tpu as pltpu
from jax.experimental.pallas import tpu_sc as plsc

BATCH = 1024
NUM_FEATURES = 768
MARGIN = 0.1
TAU = 0.02
_NEG_INF = -1e30

_NC, _NS = 2, 16
_NW = _NC * _NS
_ROWS_PER_W = BATCH // _NW
_NCHUNK = 4
_CHUNK = _ROWS_PER_W // _NCHUNK


def _gather_body(idx_hbm, table_hbm, out_hbm, idx_v, *bufs_and_sems):
    rows_v = bufs_and_sems[:_NCHUNK]
    sem_g = bufs_and_sems[_NCHUNK:2 * _NCHUNK]
    sem_w = bufs_and_sems[2 * _NCHUNK:]
    wid = lax.axis_index("s") * _NC + lax.axis_index("c")
    base = wid * _ROWS_PER_W
    pltpu.sync_copy(idx_hbm.at[pl.ds(base, _ROWS_PER_W)], idx_v)
    gathers = [
        pltpu.async_copy(
            table_hbm.at[idx_v.at[pl.ds(c * _CHUNK, _CHUNK)]], rows_v[c], sem_g[c]
        )
        for c in range(_NCHUNK)
    ]
    writes = []
    for c in range(_NCHUNK):
        gathers[c].wait()
        writes.append(
            pltpu.async_copy(
                rows_v[c], out_hbm.at[pl.ds(base + c * _CHUNK, _CHUNK)], sem_w[c]
            )
        )
    for w in writes:
        w.wait()


def _sc_gather(targets, features):
    mesh = plsc.VectorSubcoreMesh(core_axis_name="c", subcore_axis_name="s")
    k = pl.kernel(
        _gather_body,
        mesh=mesh,
        out_type=jax.ShapeDtypeStruct((BATCH, NUM_FEATURES), jnp.float32),
        scratch_types=[
            pltpu.VMEM((_ROWS_PER_W,), jnp.int32),
            *[pltpu.VMEM((_CHUNK, NUM_FEATURES), jnp.float32)] * _NCHUNK,
            *[pltpu.SemaphoreType.DMA] * (2 * _NCHUNK),
        ],
    )
    return k(targets.astype(jnp.int32), features)


_BLK = 256
_NBLK = BATCH // _BLK


def _norm_body(x_ref, xi_ref):
    x = x_ref[...]
    n = jnp.sqrt(jnp.sum(x * x, axis=1, keepdims=True))
    xi_ref[...] = (
        x * (jnp.float32(1.0 / TAU) / jnp.maximum(n, 1e-12))
    ).astype(jnp.bfloat16)


def _tc_norm(i_feats):
    return pl.pallas_call(
        _norm_body,
        grid=(_NBLK,),
        in_specs=[pl.BlockSpec((_BLK, NUM_FEATURES), lambda i: (i, 0))],
        out_specs=pl.BlockSpec((_BLK, NUM_FEATURES), lambda i: (i, 0)),
        out_shape=jax.ShapeDtypeStruct((BATCH, NUM_FEATURES), jnp.bfloat16),
    )(i_feats)


def _loss_body(xi_ref, cl_ref, trow_ref, tcolb_ref, out_ref,
               sumE_ref, sumES_ref, negr_ref, acc_ref, ssum_ref):
    j = pl.program_id(0)

    @pl.when(j == 0)
    def _init():
        sumE_ref[...] = jnp.zeros((BATCH, 1), jnp.float32)
        sumES_ref[...] = jnp.zeros((BATCH, 1), jnp.float32)
        negr_ref[...] = jnp.full((BATCH, 1), _NEG_INF, jnp.float32)
        acc_ref[0, 0] = jnp.float32(0.0)
        ssum_ref[0, 0] = jnp.float32(0.0)

    st = lax.dot_general(
        xi_ref[...], cl_ref[...].astype(jnp.bfloat16), (((1,), (1,)), ((), ())),
        preferred_element_type=jnp.float32,
    )
    labels = trow_ref[...] == tcolb_ref[...]
    E = jnp.where(labels, jnp.exp(st), 0.0)
    ES = E * st
    nm = jnp.where(labels, _NEG_INF, st)

    pos2 = jnp.sum(ES, axis=0, keepdims=True) / jnp.sum(E, axis=0, keepdims=True)
    neg2 = jnp.max(nm, axis=0, keepdims=True)
    c2 = jnp.maximum(MARGIN + jnp.float32(TAU) * (neg2 - pos2), 0.0)
    acc_ref[0, 0] += jnp.sum(c2)
    ssum_ref[0, 0] += jnp.sum(st)

    sumE_ref[...] += jnp.sum(E, axis=1, keepdims=True)
    sumES_ref[...] += jnp.sum(ES, axis=1, keepdims=True)
    negr_ref[...] = jnp.maximum(negr_ref[...], jnp.max(nm, axis=1, keepdims=True))

    @pl.when(j == _NBLK - 1)
    def _final():
        pos1 = sumES_ref[...] / sumE_ref[...]
        c1 = jnp.maximum(MARGIN + jnp.float32(TAU) * (negr_ref[...] - pos1), 0.0)
        tri = acc_ref[0, 0] + jnp.sum(c1)
        center = 1.0 - ssum_ref[0, 0] * jnp.float32(TAU / (BATCH * BATCH))
        out_ref[0, 0] = tri + 0.08 * center


def _tc_loss(xi, cl, targets):
    t = targets.astype(jnp.int32)
    out = pl.pallas_call(
        _loss_body,
        grid=(_NBLK,),
        in_specs=[
            pl.BlockSpec((BATCH, NUM_FEATURES), lambda j: (0, 0)),
            pl.BlockSpec((_BLK, NUM_FEATURES), lambda j: (j, 0)),
            pl.BlockSpec((BATCH, 1), lambda j: (0, 0)),
            pl.BlockSpec((1, _BLK), lambda j: (0, j)),
        ],
        out_specs=pl.BlockSpec((1, 1), lambda j: (0, 0), memory_space=pltpu.SMEM),
        out_shape=jax.ShapeDtypeStruct((1, 1), jnp.float32),
        scratch_shapes=[
            pltpu.VMEM((BATCH, 1), jnp.float32),
            pltpu.VMEM((BATCH, 1), jnp.float32),
            pltpu.VMEM((BATCH, 1), jnp.float32),
            pltpu.SMEM((1, 1), jnp.float32),
            pltpu.SMEM((1, 1), jnp.float32),
        ],
    )(xi, cl, t.reshape(BATCH, 1), t.reshape(1, BATCH))
    return out[0, 0]


def kernel(i_feats, targets, features):
    cl = _sc_gather(targets, features)
    xi = _tc_norm(i_feats)
    return _tc_loss(xi, cl, targets)
